# baseline (speedup 1.0000x reference)
import functools

import jax
import jax.numpy as jnp
from jax import lax
from jax.experimental import pallas as pl
from jax.experimental.pallas import tpu as pltpu
from jax.experimental.pallas import tpu_sc as plsc


@functools.lru_cache(maxsize=None)
def _make_gather(num_rows, dim, pad_dim, n_r, n_c):
    info = plsc.get_sparse_core_info()
    nc, ns, nl = info.num_cores, info.num_subcores, info.num_lanes
    nw = nc * ns
    assert n_r % (nw * nl) == 0 and n_c <= 2 * nl
    r_per_w = n_r // nw
    b_per_w = r_per_w * n_c
    batch = n_r * n_c
    chunk = 256
    while b_per_w % chunk:
        chunk //= 2
    n_chunks = b_per_w // chunk

    mesh = plsc.VectorSubcoreMesh(core_axis_name="c", subcore_axis_name="s")

    @functools.partial(
        pl.kernel,
        mesh=mesh,
        out_type=jax.ShapeDtypeStruct((batch, dim), jnp.float32),
        scratch_types=[
            pltpu.VMEM((r_per_w, n_c), jnp.int32),
            pltpu.VMEM((b_per_w,), jnp.int32),
            pltpu.VMEM((2, chunk, pad_dim), jnp.float32),
            pltpu.SemaphoreType.DMA,
            pltpu.SemaphoreType.DMA,
        ],
        compiler_params=pltpu.CompilerParams(
            use_tc_tiling_on_sc=False, needs_layout_passes=False),
    )
    def gather(idx_hbm, table_hbm, out_hbm, idx_blk, idx_v, rows_v, gsem, ssem):
        wid = lax.axis_index("s") * nc + lax.axis_index("c")
        r0 = wid * r_per_w
        base = wid * b_per_w
        pltpu.sync_copy(idx_hbm.at[pl.ds(r0, r_per_w), :], idx_blk)
        iota = lax.iota(jnp.int32, nl)
        tail = n_c - nl
        tail_mask = iota < tail

        def flatten_row(r, _):
            r_vec = iota * 0 + r
            k0 = r * n_c
            v1 = plsc.load_gather(idx_blk, [r_vec, iota])
            plsc.store_scatter(idx_v, [iota + k0], v1)
            if tail > 0:
                v2 = plsc.load_gather(
                    idx_blk, [r_vec, iota + nl], mask=tail_mask)
                plsc.store_scatter(
                    idx_v, [iota + (k0 + nl)], v2, mask=tail_mask)
            return _

        lax.fori_loop(0, r_per_w, flatten_row, None)

        def gather_start(cnk):
            return pltpu.async_copy(
                table_hbm.at[idx_v.at[pl.ds(cnk * chunk, chunk)]],
                rows_v.at[cnk % 2], gsem)

        def store_start(cnk):
            return pltpu.async_copy(
                rows_v.at[cnk % 2, :, pl.ds(0, dim)],
                out_hbm.at[pl.ds(base + cnk * chunk, chunk)], ssem)

        g = gather_start(0)
        s_prev = None
        for cnk in range(n_chunks):
            g.wait()
            s = store_start(cnk)
            if cnk + 1 < n_chunks:
                if s_prev is not None:
                    s_prev.wait()
                g = gather_start(cnk + 1)
            s_prev_old, s_prev = s_prev, s
        s_prev.wait()
        if n_chunks > 1:
            s_prev_old.wait()

    return gather


def kernel(indices, weight):
    n_r, n_c = indices.shape
    dim = weight.shape[1]
    pad_dim = 128
    w128 = jnp.pad(weight, ((0, 0), (0, pad_dim - dim)))
    gather = _make_gather(weight.shape[0], dim, pad_dim, n_r, n_c)
    out = gather(indices.astype(jnp.int32), w128)
    return out.reshape(n_r, n_c, dim)


# speedup vs baseline: 1.1423x; 1.1423x over previous
import functools

import jax
import jax.numpy as jnp
from jax import lax
from jax.experimental import pallas as pl
from jax.experimental.pallas import tpu as pltpu
from jax.experimental.pallas import tpu_sc as plsc


@functools.lru_cache(maxsize=None)
def _make_gather(num_rows, dim, n_r, n_c):
    info = plsc.get_sparse_core_info()
    nc, ns, nl = info.num_cores, info.num_subcores, info.num_lanes
    nw = nc * ns
    assert n_r % (nw * nl) == 0 and n_c <= 2 * nl
    r_per_w = n_r // nw
    b_per_w = r_per_w * n_c
    chunk_r = 32
    n_chunks = r_per_w // chunk_r
    chunk = chunk_r * n_c

    mesh = plsc.VectorSubcoreMesh(core_axis_name="c", subcore_axis_name="s")

    @functools.partial(
        pl.kernel,
        mesh=mesh,
        out_type=jax.ShapeDtypeStruct((n_r, n_c, dim), jnp.float32),
        scratch_types=[
            pltpu.VMEM((r_per_w, n_c), jnp.int32),
            pltpu.VMEM((b_per_w,), jnp.int32),
            pltpu.VMEM((2, chunk, dim), jnp.float32),
            pltpu.SemaphoreType.DMA,
            pltpu.SemaphoreType.DMA,
        ],
        compiler_params=pltpu.CompilerParams(
            use_tc_tiling_on_sc=False, needs_layout_passes=False),
    )
    def gather(idx_hbm, table_hbm, out_hbm, idx_blk, idx_v, rows_v, gsem, ssem):
        wid = lax.axis_index("s") * nc + lax.axis_index("c")
        r0 = wid * r_per_w
        pltpu.sync_copy(idx_hbm.at[pl.ds(r0, r_per_w), :], idx_blk)
        iota = lax.iota(jnp.int32, nl)
        tail = n_c - nl
        tail_mask = iota < tail

        def flatten_row(r, _):
            r_vec = iota * 0 + r
            k0 = r * n_c
            v1 = plsc.load_gather(idx_blk, [r_vec, iota])
            plsc.store_scatter(idx_v, [iota + k0], v1)
            if tail > 0:
                v2 = plsc.load_gather(
                    idx_blk, [r_vec, iota + nl], mask=tail_mask)
                plsc.store_scatter(
                    idx_v, [iota + (k0 + nl)], v2, mask=tail_mask)
            return _

        lax.fori_loop(0, r_per_w, flatten_row, None)

        def gather_start(cnk):
            return pltpu.async_copy(
                table_hbm.at[idx_v.at[pl.ds(cnk * chunk, chunk)]],
                rows_v.at[cnk % 2], gsem)

        def store_start(cnk):
            buf = rows_v.at[cnk % 2]
            row_base = r0 + cnk * chunk_r
            handles = []
            for q in range(chunk_r):
                handles.append(pltpu.async_copy(
                    buf.at[pl.ds(q * n_c, n_c), :],
                    out_hbm.at[row_base + q], ssem))
            return handles

        g = gather_start(0)
        s_prev = None
        for cnk in range(n_chunks):
            g.wait()
            s = store_start(cnk)
            if cnk + 1 < n_chunks:
                if s_prev is not None:
                    for h in s_prev:
                        h.wait()
                g = gather_start(cnk + 1)
            s_prev_old, s_prev = s_prev, s
        for h in s_prev:
            h.wait()
        if n_chunks > 1:
            for h in s_prev_old:
                h.wait()

    return gather


def kernel(indices, weight):
    n_r, n_c = indices.shape
    dim = weight.shape[1]
    gather = _make_gather(weight.shape[0], dim, n_r, n_c)
    return gather(indices.astype(jnp.int32), weight)


# speedup vs baseline: 1.1649x; 1.0198x over previous
import functools

import jax
import jax.numpy as jnp
from jax import lax
from jax.experimental import pallas as pl
from jax.experimental.pallas import tpu as pltpu
from jax.experimental.pallas import tpu_sc as plsc


@functools.lru_cache(maxsize=None)
def _make_gather(num_rows, dim, batch):
    info = plsc.get_sparse_core_info()
    nc, ns = info.num_cores, info.num_subcores
    nw = nc * ns
    assert batch % nw == 0
    b_per_w = batch // nw
    chunk = 1024
    while b_per_w % chunk:
        chunk //= 2
    n_chunks = b_per_w // chunk

    mesh = plsc.VectorSubcoreMesh(core_axis_name="c", subcore_axis_name="s")

    @functools.partial(
        pl.kernel,
        mesh=mesh,
        out_type=jax.ShapeDtypeStruct((batch, dim), jnp.float32),
        scratch_types=[
            pltpu.VMEM((b_per_w,), jnp.int32),
            pltpu.VMEM((2, chunk, dim), jnp.float32),
            pltpu.SemaphoreType.DMA,
            pltpu.SemaphoreType.DMA,
        ],
        compiler_params=pltpu.CompilerParams(use_tc_tiling_on_sc=False),
    )
    def gather(idx_hbm, table_hbm, out_hbm, idx_v, rows_v, gsem, ssem):
        wid = lax.axis_index("s") * nc + lax.axis_index("c")
        base = wid * b_per_w
        pltpu.sync_copy(idx_hbm.at[pl.ds(base, b_per_w)], idx_v)

        def gather_start(c):
            return pltpu.async_copy(
                table_hbm.at[idx_v.at[pl.ds(c * chunk, chunk)]],
                rows_v.at[c % 2], gsem)

        def store_start(c):
            return pltpu.async_copy(
                rows_v.at[c % 2],
                out_hbm.at[pl.ds(base + c * chunk, chunk)], ssem)

        g = gather_start(0)
        s_prev = None
        for c in range(n_chunks):
            g.wait()
            s = store_start(c)
            if c + 1 < n_chunks:
                if s_prev is not None:
                    s_prev.wait()
                g = gather_start(c + 1)
            s_prev_old, s_prev = s_prev, s
        s_prev.wait()
        if n_chunks > 1:
            s_prev_old.wait()

    return gather


def kernel(indices, weight):
    batch = indices.shape[0] * indices.shape[1]
    flat = indices.reshape(batch).astype(jnp.int32)
    gather = _make_gather(weight.shape[0], weight.shape[1], batch)
    out = gather(flat, weight)
    return out.reshape(indices.shape + (weight.shape[1],))
